# baseline (speedup 1.0000x reference)
import functools

import jax
import jax.numpy as jnp
from jax import lax
from jax.experimental import pallas as pl
from jax.experimental.pallas import tpu as pltpu
from jax.experimental.pallas import tpu_sc as plsc


def _labels_body(C, logits_ref, gate_ref, out_ref):
    x = logits_ref[...]
    m = jnp.max(x, axis=1, keepdims=True)
    col = lax.broadcasted_iota(jnp.int32, x.shape, 1)
    idx = jnp.min(jnp.where(x == m, col, C), axis=1)
    g = gate_ref[0, 0, :]
    out_ref[0, 0, :] = jnp.where(g > 0, idx, C).reshape(1, 1, -1)[0, 0, :]


def _compute_labels(logits, gate3, nt, blk):
    C = logits.shape[1]
    nb = nt // blk
    return pl.pallas_call(
        functools.partial(_labels_body, C),
        grid=(nb,),
        in_specs=[
            pl.BlockSpec((blk, C), lambda i: (i, 0)),
            pl.BlockSpec((1, 1, blk), lambda i: (i, 0, 0)),
        ],
        out_specs=pl.BlockSpec((1, 1, blk), lambda i: (i, 0, 0)),
        out_shape=jax.ShapeDtypeStruct((nb, 1, blk), jnp.int32),
    )(logits, gate3)


def _make_sc_argmax_scatter(N, C, D, CP, row_lo):
    info = plsc.get_sparse_core_info()
    nc, ns = info.num_cores, info.num_subcores
    rows_total = N - row_lo
    rows_per_tile = rows_total // (nc * ns)
    chunk = 64
    n_chunks = rows_per_tile // chunk
    n_vec = C // 16
    tail = C - n_vec * 16

    mesh = plsc.VectorSubcoreMesh(core_axis_name="c", subcore_axis_name="s")

    @functools.partial(
        pl.kernel,
        mesh=mesh,
        out_type=[
            jax.ShapeDtypeStruct((nc, CP, D), jnp.float32),
            jax.ShapeDtypeStruct((nc, CP, D), jnp.float32),
        ],
        scratch_types=[
            pltpu.VMEM((chunk, C), jnp.float32),
            pltpu.VMEM((chunk, D), jnp.float32),
            pltpu.VMEM((chunk, D), jnp.float32),
            pltpu.VMEM((chunk,), jnp.int32),
            pltpu.VMEM((chunk,), jnp.int32),
            pltpu.VMEM_SHARED((CP, D), jnp.float32),
            pltpu.VMEM_SHARED((CP, D), jnp.float32),
        ],
    )
    def sca(logits_hbm, img_hbm, gate_hbm, zsum_hbm, zcnt_hbm, ones_hbm,
            sums_out, cnts_out,
            log_v, img_v, ones_v, gate_v, idx_v, sums_sh, cnts_sh):
        cid = lax.axis_index("c")
        sid = lax.axis_index("s")

        @pl.when(sid == 0)
        def _():
            pltpu.sync_copy(zsum_hbm, sums_sh)
            pltpu.sync_copy(zcnt_hbm, cnts_sh)

        pltpu.sync_copy(ones_hbm, ones_v)
        plsc.subcore_barrier()

        lane = lax.iota(jnp.int32, 16)
        perms = [((lane + s) & 15).reshape(16, 1) for s in (8, 4, 2, 1)]
        gdn = lax.GatherDimensionNumbers(
            offset_dims=(), collapsed_slice_dims=(0,), start_index_map=(0,))

        def _perm(v, p):
            return lax.gather(v, p, gdn, (1,),
                              mode=lax.GatherScatterMode.PROMISE_IN_BOUNDS)

        rbase = row_lo + (cid * ns + sid) * rows_per_tile

        def chunk_body(j, _):
            r0 = pl.multiple_of(rbase + j * chunk, chunk)
            pltpu.sync_copy(logits_hbm.at[pl.ds(r0, chunk)], log_v)
            pltpu.sync_copy(img_hbm.at[pl.ds(r0, chunk)], img_v)
            pltpu.sync_copy(gate_hbm.at[pl.ds(r0, chunk)], gate_v)

            def row_body(r, acc):
                best = log_v[r, pl.ds(0, 16)]
                bidx = lane
                for k in range(1, n_vec):
                    v = log_v[r, pl.ds(k * 16, 16)]
                    m = v > best
                    best = jnp.where(m, v, best)
                    bidx = jnp.where(m, lane + (k * 16), bidx)
                if tail:
                    off = C - 16
                    v = log_v[r, pl.ds(off, 16)]
                    m = v > best
                    best = jnp.where(m, v, best)
                    bidx = jnp.where(m, lane + off, bidx)
                for p in perms:
                    ov = _perm(best, p)
                    oi = _perm(bidx, p)
                    take = (ov > best) | ((ov == best) & (oi < bidx))
                    best = jnp.where(take, ov, best)
                    bidx = jnp.where(take, oi, bidx)
                rv = jnp.full((16,), r, jnp.int32)
                return tuple(
                    jnp.where(rv == lane + 16 * q, bidx, acc[q])
                    for q in range(len(acc))
                )

            ngrp = chunk // 16
            acc0 = tuple(jnp.zeros((16,), jnp.int32) for _ in range(ngrp))
            acc = lax.fori_loop(0, chunk, row_body, acc0)
            for q in range(ngrp):
                g16 = gate_v[pl.ds(q * 16, 16)]
                idx_v[pl.ds(q * 16, 16)] = jnp.where(g16 > 0, acc[q], C)
            pltpu.sync_copy(img_v, sums_sh.at[idx_v], add=True)
            pltpu.sync_copy(ones_v, cnts_sh.at[idx_v], add=True)
            return 0

        lax.fori_loop(0, n_chunks, chunk_body, 0)

        plsc.subcore_barrier()

        @pl.when(sid == 0)
        def _():
            pltpu.sync_copy(sums_sh, sums_out.at[cid])
            pltpu.sync_copy(cnts_sh, cnts_out.at[cid])

    return sca


def _make_segment_sum(nt, D, CP, chunk):
    info = plsc.get_sparse_core_info()
    nc, ns = info.num_cores, info.num_subcores
    rows_per_tile = nt // (nc * ns)
    n_chunks = rows_per_tile // chunk
    lrows = chunk // 128
    tile_lrows = rows_per_tile // 128

    mesh = plsc.VectorSubcoreMesh(core_axis_name="c", subcore_axis_name="s")

    @functools.partial(
        pl.kernel,
        mesh=mesh,
        out_type=[
            jax.ShapeDtypeStruct((nc, CP, D), jnp.float32),
            jax.ShapeDtypeStruct((nc, CP, D), jnp.float32),
        ],
        scratch_types=[
            pltpu.VMEM((tile_lrows, 128), jnp.int32),
            pltpu.VMEM((chunk, D), jnp.float32),
            pltpu.VMEM((chunk, D), jnp.float32),
            pltpu.VMEM_SHARED((CP, D), jnp.float32),
            pltpu.VMEM_SHARED((CP, D), jnp.float32),
        ],
    )
    def seg(lbl_hbm, img_hbm, zsum_hbm, zcnt_hbm, ones_hbm,
            sums_out, cnts_out, lbl_v, img_v, ones_v, sums_sh, cnts_sh):
        cid = lax.axis_index("c")
        sid = lax.axis_index("s")

        @pl.when(sid == 0)
        def _():
            pltpu.sync_copy(zsum_hbm, sums_sh)
            pltpu.sync_copy(zcnt_hbm, cnts_sh)

        pltpu.sync_copy(ones_hbm, ones_v)
        plsc.subcore_barrier()

        rbase = (cid * ns + sid) * rows_per_tile
        lb = pl.multiple_of(rbase // 128, tile_lrows)
        pltpu.sync_copy(lbl_hbm.at[pl.ds(lb, tile_lrows)], lbl_v)
        for j in range(n_chunks):
            r0 = pl.multiple_of(rbase + j * chunk, chunk)
            pltpu.sync_copy(img_hbm.at[pl.ds(r0, chunk)], img_v)
            for k in range(lrows):
                idx = lbl_v.at[j * lrows + k]
                src = img_v.at[pl.ds(k * 128, 128)]
                pltpu.sync_copy(src, sums_sh.at[idx], add=True)
                pltpu.sync_copy(ones_v.at[pl.ds(k * 128, 128)],
                                cnts_sh.at[idx], add=True)

        plsc.subcore_barrier()

        @pl.when(sid == 0)
        def _():
            pltpu.sync_copy(sums_sh, sums_out.at[cid])
            pltpu.sync_copy(cnts_sh, cnts_out.at[cid])

    return seg


def _final_body(C, sa_ref, sb_ref, ca_ref, cb_ref, text_ref, out_ref):
    s = sa_ref[0] + sa_ref[1] + sb_ref[0] + sb_ref[1]
    cnt = (ca_ref[0, :, 0] + ca_ref[1, :, 0]
           + cb_ref[0, :, 0] + cb_ref[1, :, 0])
    CP = s.shape[0]
    rows = lax.broadcasted_iota(jnp.int32, (CP,), 0)
    valid = (cnt > 0.0) & (rows < C)
    safe = jnp.where(cnt > 0.0, cnt, 1.0)
    means = s / safe[:, None]
    d = jnp.sum(means * text_ref[...], axis=1)
    num_present = jnp.sum(valid.astype(jnp.float32))
    loss = jnp.sum(jnp.where(valid, d, 0.0)) / jnp.maximum(num_present, 1.0)
    out_ref[...] = loss.reshape(1, 1)


def _finalize(sumsA, sumsB, cntsA, cntsB, text_pad, C):
    return pl.pallas_call(
        functools.partial(_final_body, C),
        out_shape=jax.ShapeDtypeStruct((1, 1), jnp.float32),
    )(sumsA, sumsB, cntsA, cntsB, text_pad)


@jax.jit
def kernel(logits, img_feats, text_norm_feats, gate_mask):
    N, C = logits.shape
    D = img_feats.shape[1]
    CP = ((C + 1 + 15) // 16) * 16
    NT = N // 2
    BLK = 2048
    CHUNK = 256

    gate_i32 = gate_mask.astype(jnp.int32)
    gate3 = gate_i32[:NT].reshape(NT // BLK, 1, BLK)

    zsum = jnp.zeros((CP, D), jnp.float32)
    zcnt = jnp.zeros((CP, D), jnp.float32)
    ones64 = jnp.ones((64, D), jnp.float32)
    ones256 = jnp.ones((CHUNK, D), jnp.float32)

    sca = _make_sc_argmax_scatter(N, C, D, CP, NT)
    sumsA, cntsA = sca(logits, img_feats, gate_i32, zsum, zcnt, ones64)

    labels3 = _compute_labels(logits, gate3, NT, BLK)
    lbl2 = labels3.reshape(NT // 128, 128)
    seg = _make_segment_sum(NT, D, CP, CHUNK)
    sumsB, cntsB = seg(lbl2, img_feats, zsum, zcnt, ones256)

    text_pad = jnp.pad(text_norm_feats, ((0, CP - C), (0, 0)))
    loss = _finalize(sumsA, sumsB, cntsA, cntsB, text_pad, C)
    return loss[0, 0]


# speedup vs baseline: 1.2508x; 1.2508x over previous
import functools

import jax
import jax.numpy as jnp
from jax import lax
from jax.experimental import pallas as pl
from jax.experimental.pallas import tpu as pltpu
from jax.experimental.pallas import tpu_sc as plsc


def _labels_body(C, logits_ref, gate_ref, out_ref):
    x = logits_ref[...]
    m = jnp.max(x, axis=1, keepdims=True)
    col = lax.broadcasted_iota(jnp.int32, x.shape, 1)
    idx = jnp.min(jnp.where(x == m, col, C), axis=1)
    g = gate_ref[0, 0, :]
    out_ref[0, 0, :] = jnp.where(g > 0, idx, C).reshape(1, 1, -1)[0, 0, :]


def _compute_labels(logits, gate3, nt, blk):
    C = logits.shape[1]
    nb = nt // blk
    return pl.pallas_call(
        functools.partial(_labels_body, C),
        grid=(nb,),
        in_specs=[
            pl.BlockSpec((blk, C), lambda i: (i, 0)),
            pl.BlockSpec((1, 1, blk), lambda i: (i, 0, 0)),
        ],
        out_specs=pl.BlockSpec((1, 1, blk), lambda i: (i, 0, 0)),
        out_shape=jax.ShapeDtypeStruct((nb, 1, blk), jnp.int32),
    )(logits, gate3)


def _make_sc_argmax_scatter(N, C, D, CP, row_lo):
    info = plsc.get_sparse_core_info()
    nc, ns = info.num_cores, info.num_subcores
    rows_total = N - row_lo
    rows_per_tile = rows_total // (nc * ns)
    chunk = 64
    n_chunks = rows_per_tile // chunk
    n_vec = C // 16
    tail = C - n_vec * 16

    mesh = plsc.VectorSubcoreMesh(core_axis_name="c", subcore_axis_name="s")

    @functools.partial(
        pl.kernel,
        mesh=mesh,
        out_type=[
            jax.ShapeDtypeStruct((nc, CP, D), jnp.float32),
            jax.ShapeDtypeStruct((nc, CP, D), jnp.float32),
        ],
        scratch_types=[
            pltpu.VMEM((chunk, C), jnp.float32),
            pltpu.VMEM((chunk, D), jnp.float32),
            pltpu.VMEM((chunk, D), jnp.float32),
            pltpu.VMEM((chunk,), jnp.int32),
            pltpu.VMEM((chunk,), jnp.int32),
            pltpu.VMEM_SHARED((CP, D), jnp.float32),
            pltpu.VMEM_SHARED((CP, D), jnp.float32),
        ],
    )
    def sca(logits_hbm, img_hbm, gate_hbm, zsum_hbm, zcnt_hbm, ones_hbm,
            sums_out, cnts_out,
            log_v, img_v, ones_v, gate_v, idx_v, sums_sh, cnts_sh):
        cid = lax.axis_index("c")
        sid = lax.axis_index("s")

        @pl.when(sid == 0)
        def _():
            pltpu.sync_copy(zsum_hbm, sums_sh)
            pltpu.sync_copy(zcnt_hbm, cnts_sh)

        pltpu.sync_copy(ones_hbm, ones_v)
        plsc.subcore_barrier()

        lane = lax.iota(jnp.int32, 16)
        perms = [((lane + s) & 15).reshape(16, 1) for s in (8, 4, 2, 1)]
        gdn = lax.GatherDimensionNumbers(
            offset_dims=(), collapsed_slice_dims=(0,), start_index_map=(0,))

        def _perm(v, p):
            return lax.gather(v, p, gdn, (1,),
                              mode=lax.GatherScatterMode.PROMISE_IN_BOUNDS)

        rbase = row_lo + (cid * ns + sid) * rows_per_tile

        def chunk_body(j, _):
            r0 = pl.multiple_of(rbase + j * chunk, chunk)
            pltpu.sync_copy(logits_hbm.at[pl.ds(r0, chunk)], log_v)
            pltpu.sync_copy(img_hbm.at[pl.ds(r0, chunk)], img_v)
            pltpu.sync_copy(gate_hbm.at[pl.ds(r0, chunk)], gate_v)

            def row_body(r, acc):
                best = log_v[r, pl.ds(0, 16)]
                bidx = lane
                for k in range(1, n_vec):
                    v = log_v[r, pl.ds(k * 16, 16)]
                    m = v > best
                    best = jnp.where(m, v, best)
                    bidx = jnp.where(m, lane + (k * 16), bidx)
                if tail:
                    off = C - 16
                    v = log_v[r, pl.ds(off, 16)]
                    m = v > best
                    best = jnp.where(m, v, best)
                    bidx = jnp.where(m, lane + off, bidx)
                for p in perms:
                    ov = _perm(best, p)
                    oi = _perm(bidx, p)
                    take = (ov > best) | ((ov == best) & (oi < bidx))
                    best = jnp.where(take, ov, best)
                    bidx = jnp.where(take, oi, bidx)
                rv = jnp.full((16,), r, jnp.int32)
                return tuple(
                    jnp.where(rv == lane + 16 * q, bidx, acc[q])
                    for q in range(len(acc))
                )

            ngrp = chunk // 16
            acc0 = tuple(jnp.zeros((16,), jnp.int32) for _ in range(ngrp))
            acc = lax.fori_loop(0, chunk, row_body, acc0)
            for q in range(ngrp):
                g16 = gate_v[pl.ds(q * 16, 16)]
                idx_v[pl.ds(q * 16, 16)] = jnp.where(g16 > 0, acc[q], C)
            pltpu.sync_copy(img_v, sums_sh.at[idx_v], add=True)
            pltpu.sync_copy(ones_v, cnts_sh.at[idx_v], add=True)
            return 0

        lax.fori_loop(0, n_chunks, chunk_body, 0)

        plsc.subcore_barrier()

        @pl.when(sid == 0)
        def _():
            pltpu.sync_copy(sums_sh, sums_out.at[cid])
            pltpu.sync_copy(cnts_sh, cnts_out.at[cid])

    return sca


def _make_segment_sum(nt, D, CP, chunk):
    info = plsc.get_sparse_core_info()
    nc, ns = info.num_cores, info.num_subcores
    rows_per_tile = nt // (nc * ns)
    n_chunks = rows_per_tile // chunk
    lrows = chunk // 128

    mesh = plsc.VectorSubcoreMesh(core_axis_name="c", subcore_axis_name="s")

    @functools.partial(
        pl.kernel,
        mesh=mesh,
        out_type=[
            jax.ShapeDtypeStruct((nc, CP, D), jnp.float32),
            jax.ShapeDtypeStruct((nc, CP, D), jnp.float32),
        ],
        scratch_types=[
            [pltpu.VMEM((128,), jnp.int32) for _ in range(lrows)],
            pltpu.VMEM((chunk, D), jnp.float32),
            pltpu.VMEM((chunk, D), jnp.float32),
            pltpu.VMEM_SHARED((CP, D), jnp.float32),
            pltpu.VMEM_SHARED((CP, D), jnp.float32),
        ],
    )
    def seg(lbl_hbm, img_hbm, zsum_hbm, zcnt_hbm, ones_hbm,
            sums_out, cnts_out, lbl_vs, img_v, ones_v, sums_sh, cnts_sh):
        cid = lax.axis_index("c")
        sid = lax.axis_index("s")

        @pl.when(sid == 0)
        def _():
            pltpu.sync_copy(zsum_hbm, sums_sh)
            pltpu.sync_copy(zcnt_hbm, cnts_sh)

        pltpu.sync_copy(ones_hbm, ones_v)
        plsc.subcore_barrier()

        rbase = (cid * ns + sid) * rows_per_tile
        for j in range(n_chunks):
            r0 = pl.multiple_of(rbase + j * chunk, chunk)
            pltpu.sync_copy(img_hbm.at[pl.ds(r0, chunk)], img_v)
            for k in range(lrows):
                rk = pl.multiple_of(r0 + k * 128, 128)
                pltpu.sync_copy(lbl_hbm.at[pl.ds(rk, 128)], lbl_vs[k])
            for k in range(lrows):
                src = img_v.at[pl.ds(k * 128, 128)]
                pltpu.sync_copy(src, sums_sh.at[lbl_vs[k]], add=True)
                pltpu.sync_copy(ones_v.at[pl.ds(k * 128, 128)],
                                cnts_sh.at[lbl_vs[k]], add=True)

        plsc.subcore_barrier()

        @pl.when(sid == 0)
        def _():
            pltpu.sync_copy(sums_sh, sums_out.at[cid])
            pltpu.sync_copy(cnts_sh, cnts_out.at[cid])

    return seg


def _final_body(C, sa_ref, sb_ref, ca_ref, cb_ref, text_ref, out_ref):
    s = sa_ref[0] + sa_ref[1] + sb_ref[0] + sb_ref[1]
    cnt = (ca_ref[0, :, 0] + ca_ref[1, :, 0]
           + cb_ref[0, :, 0] + cb_ref[1, :, 0])
    CP = s.shape[0]
    rows = lax.broadcasted_iota(jnp.int32, (CP,), 0)
    valid = (cnt > 0.0) & (rows < C)
    safe = jnp.where(cnt > 0.0, cnt, 1.0)
    means = s / safe[:, None]
    d = jnp.sum(means * text_ref[...], axis=1)
    num_present = jnp.sum(valid.astype(jnp.float32))
    loss = jnp.sum(jnp.where(valid, d, 0.0)) / jnp.maximum(num_present, 1.0)
    out_ref[...] = loss.reshape(1, 1)


def _finalize(sumsA, sumsB, cntsA, cntsB, text_pad, C):
    return pl.pallas_call(
        functools.partial(_final_body, C),
        out_shape=jax.ShapeDtypeStruct((1, 1), jnp.float32),
    )(sumsA, sumsB, cntsA, cntsB, text_pad)


@jax.jit
def kernel(logits, img_feats, text_norm_feats, gate_mask):
    N, C = logits.shape
    D = img_feats.shape[1]
    CP = ((C + 1 + 15) // 16) * 16
    NT = (3 * N) // 4
    BLK = 2048
    CHUNK = 256

    gate_i32 = gate_mask.astype(jnp.int32)
    gate3 = gate_i32[:NT].reshape(NT // BLK, 1, BLK)

    zsum = jnp.zeros((CP, D), jnp.float32)
    zcnt = jnp.zeros((CP, D), jnp.float32)
    ones64 = jnp.ones((64, D), jnp.float32)
    ones256 = jnp.ones((CHUNK, D), jnp.float32)

    sca = _make_sc_argmax_scatter(N, C, D, CP, NT)
    sumsA, cntsA = sca(logits, img_feats, gate_i32, zsum, zcnt, ones64)

    labels3 = _compute_labels(logits, gate3, NT, BLK)
    lbl1 = labels3.reshape(NT)
    seg = _make_segment_sum(NT, D, CP, CHUNK)
    sumsB, cntsB = seg(lbl1, img_feats, zsum, zcnt, ones256)

    text_pad = jnp.pad(text_norm_feats, ((0, CP - C), (0, 0)))
    loss = _finalize(sumsA, sumsB, cntsA, cntsB, text_pad, C)
    return loss[0, 0]


# speedup vs baseline: 1.3879x; 1.1096x over previous
import functools

import jax
import jax.numpy as jnp
from jax import lax
from jax.experimental import pallas as pl
from jax.experimental.pallas import tpu as pltpu
from jax.experimental.pallas import tpu_sc as plsc


def _labels_body(C, logits_ref, gate_ref, out_ref):
    x = logits_ref[...]
    m = jnp.max(x, axis=1, keepdims=True)
    col = lax.broadcasted_iota(jnp.int32, x.shape, 1)
    idx = jnp.min(jnp.where(x == m, col, C), axis=1)
    g = gate_ref[0, 0, :]
    out_ref[0, 0, :] = jnp.where(g > 0, idx, C).reshape(1, 1, -1)[0, 0, :]


def _compute_labels(logits, gate3, nt, blk):
    C = logits.shape[1]
    nb = nt // blk
    return pl.pallas_call(
        functools.partial(_labels_body, C),
        grid=(nb,),
        in_specs=[
            pl.BlockSpec((blk, C), lambda i: (i, 0)),
            pl.BlockSpec((1, 1, blk), lambda i: (i, 0, 0)),
        ],
        out_specs=pl.BlockSpec((1, 1, blk), lambda i: (i, 0, 0)),
        out_shape=jax.ShapeDtypeStruct((nb, 1, blk), jnp.int32),
    )(logits, gate3)


def _make_sc_argmax_scatter(N, C, D, CP, row_lo):
    info = plsc.get_sparse_core_info()
    nc, ns = info.num_cores, info.num_subcores
    rows_total = N - row_lo
    rows_per_tile = rows_total // (nc * ns)
    chunk = 64
    n_chunks = rows_per_tile // chunk
    n_vec = C // 16
    tail = C - n_vec * 16

    mesh = plsc.VectorSubcoreMesh(core_axis_name="c", subcore_axis_name="s")

    @functools.partial(
        pl.kernel,
        mesh=mesh,
        out_type=[
            jax.ShapeDtypeStruct((nc, CP, D), jnp.float32),
            jax.ShapeDtypeStruct((nc, CP, D), jnp.float32),
        ],
        scratch_types=[
            pltpu.VMEM((chunk, C), jnp.float32),
            pltpu.VMEM((chunk, D), jnp.float32),
            pltpu.VMEM((chunk, D), jnp.float32),
            pltpu.VMEM((chunk,), jnp.int32),
            pltpu.VMEM((chunk,), jnp.int32),
            pltpu.VMEM_SHARED((CP, D), jnp.float32),
            pltpu.VMEM_SHARED((CP, D), jnp.float32),
        ],
    )
    def sca(logits_hbm, img_hbm, gate_hbm, zsum_hbm, zcnt_hbm, ones_hbm,
            sums_out, cnts_out,
            log_v, img_v, ones_v, gate_v, idx_v, sums_sh, cnts_sh):
        cid = lax.axis_index("c")
        sid = lax.axis_index("s")

        @pl.when(sid == 0)
        def _():
            pltpu.sync_copy(zsum_hbm, sums_sh)
            pltpu.sync_copy(zcnt_hbm, cnts_sh)

        pltpu.sync_copy(ones_hbm, ones_v)
        plsc.subcore_barrier()

        lane = lax.iota(jnp.int32, 16)
        perms = [((lane + s) & 15).reshape(16, 1) for s in (8, 4, 2, 1)]
        gdn = lax.GatherDimensionNumbers(
            offset_dims=(), collapsed_slice_dims=(0,), start_index_map=(0,))

        def _perm(v, p):
            return lax.gather(v, p, gdn, (1,),
                              mode=lax.GatherScatterMode.PROMISE_IN_BOUNDS)

        rbase = row_lo + (cid * ns + sid) * rows_per_tile

        def chunk_body(j, _):
            r0 = pl.multiple_of(rbase + j * chunk, chunk)
            pltpu.sync_copy(logits_hbm.at[pl.ds(r0, chunk)], log_v)
            pltpu.sync_copy(img_hbm.at[pl.ds(r0, chunk)], img_v)
            pltpu.sync_copy(gate_hbm.at[pl.ds(r0, chunk)], gate_v)

            def row_body(r, acc):
                best = log_v[r, pl.ds(0, 16)]
                bidx = lane
                for k in range(1, n_vec):
                    v = log_v[r, pl.ds(k * 16, 16)]
                    m = v > best
                    best = jnp.where(m, v, best)
                    bidx = jnp.where(m, lane + (k * 16), bidx)
                if tail:
                    off = C - 16
                    v = log_v[r, pl.ds(off, 16)]
                    m = v > best
                    best = jnp.where(m, v, best)
                    bidx = jnp.where(m, lane + off, bidx)
                for p in perms:
                    ov = _perm(best, p)
                    oi = _perm(bidx, p)
                    take = (ov > best) | ((ov == best) & (oi < bidx))
                    best = jnp.where(take, ov, best)
                    bidx = jnp.where(take, oi, bidx)
                rv = jnp.full((16,), r, jnp.int32)
                return tuple(
                    jnp.where(rv == lane + 16 * q, bidx, acc[q])
                    for q in range(len(acc))
                )

            ngrp = chunk // 16
            acc0 = tuple(jnp.zeros((16,), jnp.int32) for _ in range(ngrp))
            acc = lax.fori_loop(0, chunk, row_body, acc0)
            for q in range(ngrp):
                g16 = gate_v[pl.ds(q * 16, 16)]
                idx_v[pl.ds(q * 16, 16)] = jnp.where(g16 > 0, acc[q], C)
            pltpu.sync_copy(img_v, sums_sh.at[idx_v], add=True)
            pltpu.sync_copy(ones_v, cnts_sh.at[idx_v], add=True)
            return 0

        lax.fori_loop(0, n_chunks, chunk_body, 0)

        plsc.subcore_barrier()

        @pl.when(sid == 0)
        def _():
            pltpu.sync_copy(sums_sh, sums_out.at[cid])
            pltpu.sync_copy(cnts_sh, cnts_out.at[cid])

    return sca


def _make_segment_sum(nt, D, CP, chunk):
    info = plsc.get_sparse_core_info()
    nc, ns = info.num_cores, info.num_subcores
    rows_per_tile = nt // (nc * ns)
    n_chunks = rows_per_tile // chunk
    lrows = chunk // 128

    mesh = plsc.VectorSubcoreMesh(core_axis_name="c", subcore_axis_name="s")

    @functools.partial(
        pl.kernel,
        mesh=mesh,
        out_type=[
            jax.ShapeDtypeStruct((nc, CP, D), jnp.float32),
            jax.ShapeDtypeStruct((nc, CP, D), jnp.float32),
        ],
        scratch_types=[
            [pltpu.VMEM((128,), jnp.int32) for _ in range(lrows)],
            pltpu.VMEM((chunk, D), jnp.float32),
            pltpu.VMEM((chunk, D), jnp.float32),
            pltpu.VMEM_SHARED((CP, D), jnp.float32),
            pltpu.VMEM_SHARED((CP, D), jnp.float32),
        ],
    )
    def seg(lbl_hbm, img_hbm, zsum_hbm, zcnt_hbm, ones_hbm,
            sums_out, cnts_out, lbl_vs, img_v, ones_v, sums_sh, cnts_sh):
        cid = lax.axis_index("c")
        sid = lax.axis_index("s")

        @pl.when(sid == 0)
        def _():
            pltpu.sync_copy(zsum_hbm, sums_sh)
            pltpu.sync_copy(zcnt_hbm, cnts_sh)

        pltpu.sync_copy(ones_hbm, ones_v)
        plsc.subcore_barrier()

        rbase = (cid * ns + sid) * rows_per_tile
        for j in range(n_chunks):
            r0 = pl.multiple_of(rbase + j * chunk, chunk)
            pltpu.sync_copy(img_hbm.at[pl.ds(r0, chunk)], img_v)
            for k in range(lrows):
                rk = pl.multiple_of(r0 + k * 128, 128)
                pltpu.sync_copy(lbl_hbm.at[pl.ds(rk, 128)], lbl_vs[k])
            for k in range(lrows):
                src = img_v.at[pl.ds(k * 128, 128)]
                pltpu.sync_copy(src, sums_sh.at[lbl_vs[k]], add=True)
                pltpu.sync_copy(ones_v.at[pl.ds(k * 128, 128)],
                                cnts_sh.at[lbl_vs[k]], add=True)

        plsc.subcore_barrier()

        @pl.when(sid == 0)
        def _():
            pltpu.sync_copy(sums_sh, sums_out.at[cid])
            pltpu.sync_copy(cnts_sh, cnts_out.at[cid])

    return seg


def _final_body_b(C, sb_ref, cb_ref, text_ref, out_ref):
    s = sb_ref[0] + sb_ref[1]
    cnt = cb_ref[0, :, 0] + cb_ref[1, :, 0]
    CP = s.shape[0]
    rows = lax.broadcasted_iota(jnp.int32, (CP,), 0)
    valid = (cnt > 0.0) & (rows < C)
    safe = jnp.where(cnt > 0.0, cnt, 1.0)
    means = s / safe[:, None]
    d = jnp.sum(means * text_ref[...], axis=1)
    num_present = jnp.sum(valid.astype(jnp.float32))
    loss = jnp.sum(jnp.where(valid, d, 0.0)) / jnp.maximum(num_present, 1.0)
    out_ref[...] = loss.reshape(1, 1)


def _final_body(C, sa_ref, sb_ref, ca_ref, cb_ref, text_ref, out_ref):
    s = sa_ref[0] + sa_ref[1] + sb_ref[0] + sb_ref[1]
    cnt = (ca_ref[0, :, 0] + ca_ref[1, :, 0]
           + cb_ref[0, :, 0] + cb_ref[1, :, 0])
    CP = s.shape[0]
    rows = lax.broadcasted_iota(jnp.int32, (CP,), 0)
    valid = (cnt > 0.0) & (rows < C)
    safe = jnp.where(cnt > 0.0, cnt, 1.0)
    means = s / safe[:, None]
    d = jnp.sum(means * text_ref[...], axis=1)
    num_present = jnp.sum(valid.astype(jnp.float32))
    loss = jnp.sum(jnp.where(valid, d, 0.0)) / jnp.maximum(num_present, 1.0)
    out_ref[...] = loss.reshape(1, 1)


def _finalize(sumsA, sumsB, cntsA, cntsB, text_pad, C):
    if sumsA is None:
        return pl.pallas_call(
            functools.partial(_final_body_b, C),
            out_shape=jax.ShapeDtypeStruct((1, 1), jnp.float32),
        )(sumsB, cntsB, text_pad)
    return pl.pallas_call(
        functools.partial(_final_body, C),
        out_shape=jax.ShapeDtypeStruct((1, 1), jnp.float32),
    )(sumsA, sumsB, cntsA, cntsB, text_pad)


@jax.jit
def kernel(logits, img_feats, text_norm_feats, gate_mask):
    N, C = logits.shape
    D = img_feats.shape[1]
    CP = ((C + 1 + 15) // 16) * 16
    NT = N
    BLK = 2048
    CHUNK = 256

    gate_i32 = gate_mask.astype(jnp.int32)
    gate3 = gate_i32[:NT].reshape(NT // BLK, 1, BLK)

    zsum = jnp.zeros((CP, D), jnp.float32)
    zcnt = jnp.zeros((CP, D), jnp.float32)
    ones256 = jnp.ones((CHUNK, D), jnp.float32)

    if NT < N:
        ones64 = jnp.ones((64, D), jnp.float32)
        sca = _make_sc_argmax_scatter(N, C, D, CP, NT)
        sumsA, cntsA = sca(logits, img_feats, gate_i32, zsum, zcnt, ones64)
    else:
        sumsA = cntsA = None

    labels3 = _compute_labels(logits, gate3, NT, BLK)
    lbl1 = labels3.reshape(NT)
    seg = _make_segment_sum(NT, D, CP, CHUNK)
    sumsB, cntsB = seg(lbl1, img_feats, zsum, zcnt, ones256)

    text_pad = jnp.pad(text_norm_feats, ((0, CP - C), (0, 0)))
    loss = _finalize(sumsA, sumsB, cntsA, cntsB, text_pad, C)
    return loss[0, 0]


# speedup vs baseline: 1.4310x; 1.0311x over previous
import functools

import jax
import jax.numpy as jnp
from jax import lax
from jax.experimental import pallas as pl
from jax.experimental.pallas import tpu as pltpu
from jax.experimental.pallas import tpu_sc as plsc


def _labels_body(C, logits_ref, gate_ref, out_ref):
    x = logits_ref[...]
    m = jnp.max(x, axis=1, keepdims=True)
    col = lax.broadcasted_iota(jnp.int32, x.shape, 1)
    idx = jnp.min(jnp.where(x == m, col, C), axis=1)
    g = gate_ref[0, 0, :]
    out_ref[0, 0, :] = jnp.where(g > 0, idx, C).reshape(1, 1, -1)[0, 0, :]


def _compute_labels(logits, gate3, nt, blk, row_off=0):
    C = logits.shape[1]
    nb = nt // blk
    ob = row_off // blk
    return pl.pallas_call(
        functools.partial(_labels_body, C),
        grid=(nb,),
        in_specs=[
            pl.BlockSpec((blk, C), lambda i: (i + ob, 0)),
            pl.BlockSpec((1, 1, blk), lambda i: (i, 0, 0)),
        ],
        out_specs=pl.BlockSpec((1, 1, blk), lambda i: (i, 0, 0)),
        out_shape=jax.ShapeDtypeStruct((nb, 1, blk), jnp.int32),
    )(logits, gate3)


def _make_sc_argmax_scatter(N, C, D, CP, row_lo):
    info = plsc.get_sparse_core_info()
    nc, ns = info.num_cores, info.num_subcores
    rows_total = N - row_lo
    rows_per_tile = rows_total // (nc * ns)
    chunk = 64
    n_chunks = rows_per_tile // chunk
    n_vec = C // 16
    tail = C - n_vec * 16

    mesh = plsc.VectorSubcoreMesh(core_axis_name="c", subcore_axis_name="s")

    @functools.partial(
        pl.kernel,
        mesh=mesh,
        out_type=[
            jax.ShapeDtypeStruct((nc, CP, D), jnp.float32),
            jax.ShapeDtypeStruct((nc, CP, D), jnp.float32),
        ],
        scratch_types=[
            pltpu.VMEM((chunk, C), jnp.float32),
            pltpu.VMEM((chunk, D), jnp.float32),
            pltpu.VMEM((chunk, D), jnp.float32),
            pltpu.VMEM((chunk,), jnp.int32),
            pltpu.VMEM((chunk,), jnp.int32),
            pltpu.VMEM_SHARED((CP, D), jnp.float32),
            pltpu.VMEM_SHARED((CP, D), jnp.float32),
        ],
    )
    def sca(logits_hbm, img_hbm, gate_hbm, zsum_hbm, zcnt_hbm, ones_hbm,
            sums_out, cnts_out,
            log_v, img_v, ones_v, gate_v, idx_v, sums_sh, cnts_sh):
        cid = lax.axis_index("c")
        sid = lax.axis_index("s")

        @pl.when(sid == 0)
        def _():
            pltpu.sync_copy(zsum_hbm, sums_sh)
            pltpu.sync_copy(zcnt_hbm, cnts_sh)

        pltpu.sync_copy(ones_hbm, ones_v)
        plsc.subcore_barrier()

        lane = lax.iota(jnp.int32, 16)
        perms = [((lane + s) & 15).reshape(16, 1) for s in (8, 4, 2, 1)]
        gdn = lax.GatherDimensionNumbers(
            offset_dims=(), collapsed_slice_dims=(0,), start_index_map=(0,))

        def _perm(v, p):
            return lax.gather(v, p, gdn, (1,),
                              mode=lax.GatherScatterMode.PROMISE_IN_BOUNDS)

        rbase = row_lo + (cid * ns + sid) * rows_per_tile

        def chunk_body(j, _):
            r0 = pl.multiple_of(rbase + j * chunk, chunk)
            pltpu.sync_copy(logits_hbm.at[pl.ds(r0, chunk)], log_v)
            pltpu.sync_copy(img_hbm.at[pl.ds(r0, chunk)], img_v)
            pltpu.sync_copy(gate_hbm.at[pl.ds(r0, chunk)], gate_v)

            def row_body(r, acc):
                best = log_v[r, pl.ds(0, 16)]
                bidx = lane
                for k in range(1, n_vec):
                    v = log_v[r, pl.ds(k * 16, 16)]
                    m = v > best
                    best = jnp.where(m, v, best)
                    bidx = jnp.where(m, lane + (k * 16), bidx)
                if tail:
                    off = C - 16
                    v = log_v[r, pl.ds(off, 16)]
                    m = v > best
                    best = jnp.where(m, v, best)
                    bidx = jnp.where(m, lane + off, bidx)
                for p in perms:
                    ov = _perm(best, p)
                    oi = _perm(bidx, p)
                    take = (ov > best) | ((ov == best) & (oi < bidx))
                    best = jnp.where(take, ov, best)
                    bidx = jnp.where(take, oi, bidx)
                rv = jnp.full((16,), r, jnp.int32)
                return tuple(
                    jnp.where(rv == lane + 16 * q, bidx, acc[q])
                    for q in range(len(acc))
                )

            ngrp = chunk // 16
            acc0 = tuple(jnp.zeros((16,), jnp.int32) for _ in range(ngrp))
            acc = lax.fori_loop(0, chunk, row_body, acc0)
            for q in range(ngrp):
                g16 = gate_v[pl.ds(q * 16, 16)]
                idx_v[pl.ds(q * 16, 16)] = jnp.where(g16 > 0, acc[q], C)
            pltpu.sync_copy(img_v, sums_sh.at[idx_v], add=True)
            pltpu.sync_copy(ones_v, cnts_sh.at[idx_v], add=True)
            return 0

        lax.fori_loop(0, n_chunks, chunk_body, 0)

        plsc.subcore_barrier()

        @pl.when(sid == 0)
        def _():
            pltpu.sync_copy(sums_sh, sums_out.at[cid])
            pltpu.sync_copy(cnts_sh, cnts_out.at[cid])

    return sca


def _make_segment_sum(nt, D, CP, chunk, row_off=0):
    info = plsc.get_sparse_core_info()
    nc, ns = info.num_cores, info.num_subcores
    rows_per_tile = nt // (nc * ns)
    n_chunks = rows_per_tile // chunk
    lrows = chunk // 128

    mesh = plsc.VectorSubcoreMesh(core_axis_name="c", subcore_axis_name="s")

    @functools.partial(
        pl.kernel,
        mesh=mesh,
        out_type=[
            jax.ShapeDtypeStruct((nc, CP, D), jnp.float32),
            jax.ShapeDtypeStruct((nc, CP, D), jnp.float32),
        ],
        scratch_types=[
            [pltpu.VMEM((128,), jnp.int32) for _ in range(lrows)],
            pltpu.VMEM((chunk, D), jnp.float32),
            pltpu.VMEM((chunk, D), jnp.float32),
            pltpu.VMEM_SHARED((CP, D), jnp.float32),
            pltpu.VMEM_SHARED((CP, D), jnp.float32),
        ],
    )
    def seg(lbl_hbm, img_hbm, zsum_hbm, zcnt_hbm, ones_hbm,
            sums_out, cnts_out, lbl_vs, img_v, ones_v, sums_sh, cnts_sh):
        cid = lax.axis_index("c")
        sid = lax.axis_index("s")

        @pl.when(sid == 0)
        def _():
            pltpu.sync_copy(zsum_hbm, sums_sh)
            pltpu.sync_copy(zcnt_hbm, cnts_sh)

        pltpu.sync_copy(ones_hbm, ones_v)
        plsc.subcore_barrier()

        lbase = (cid * ns + sid) * rows_per_tile
        for j in range(n_chunks):
            l0 = lbase + j * chunk
            r0 = pl.multiple_of(row_off + l0, chunk)
            pltpu.sync_copy(img_hbm.at[pl.ds(r0, chunk)], img_v)
            for k in range(lrows):
                rk = pl.multiple_of(l0 + k * 128, 128)
                pltpu.sync_copy(lbl_hbm.at[pl.ds(rk, 128)], lbl_vs[k])
            for k in range(lrows):
                src = img_v.at[pl.ds(k * 128, 128)]
                pltpu.sync_copy(src, sums_sh.at[lbl_vs[k]], add=True)
                pltpu.sync_copy(ones_v.at[pl.ds(k * 128, 128)],
                                cnts_sh.at[lbl_vs[k]], add=True)

        plsc.subcore_barrier()

        @pl.when(sid == 0)
        def _():
            pltpu.sync_copy(sums_sh, sums_out.at[cid])
            pltpu.sync_copy(cnts_sh, cnts_out.at[cid])

    return seg


def _final_body_b(C, sb_ref, cb_ref, text_ref, out_ref):
    s = sb_ref[0] + sb_ref[1]
    cnt = cb_ref[0, :, 0] + cb_ref[1, :, 0]
    CP = s.shape[0]
    rows = lax.broadcasted_iota(jnp.int32, (CP,), 0)
    valid = (cnt > 0.0) & (rows < C)
    safe = jnp.where(cnt > 0.0, cnt, 1.0)
    means = s / safe[:, None]
    d = jnp.sum(means * text_ref[...], axis=1)
    num_present = jnp.sum(valid.astype(jnp.float32))
    loss = jnp.sum(jnp.where(valid, d, 0.0)) / jnp.maximum(num_present, 1.0)
    out_ref[...] = loss.reshape(1, 1)


def _final_body(C, sa_ref, sb_ref, ca_ref, cb_ref, text_ref, out_ref):
    s = sa_ref[0] + sa_ref[1] + sb_ref[0] + sb_ref[1]
    cnt = (ca_ref[0, :, 0] + ca_ref[1, :, 0]
           + cb_ref[0, :, 0] + cb_ref[1, :, 0])
    CP = s.shape[0]
    rows = lax.broadcasted_iota(jnp.int32, (CP,), 0)
    valid = (cnt > 0.0) & (rows < C)
    safe = jnp.where(cnt > 0.0, cnt, 1.0)
    means = s / safe[:, None]
    d = jnp.sum(means * text_ref[...], axis=1)
    num_present = jnp.sum(valid.astype(jnp.float32))
    loss = jnp.sum(jnp.where(valid, d, 0.0)) / jnp.maximum(num_present, 1.0)
    out_ref[...] = loss.reshape(1, 1)


def _finalize(sumsA, sumsB, cntsA, cntsB, text_pad, C):
    if sumsA is None:
        return pl.pallas_call(
            functools.partial(_final_body_b, C),
            out_shape=jax.ShapeDtypeStruct((1, 1), jnp.float32),
        )(sumsB, cntsB, text_pad)
    return pl.pallas_call(
        functools.partial(_final_body, C),
        out_shape=jax.ShapeDtypeStruct((1, 1), jnp.float32),
    )(sumsA, sumsB, cntsA, cntsB, text_pad)


@jax.jit
def kernel(logits, img_feats, text_norm_feats, gate_mask):
    N, C = logits.shape
    D = img_feats.shape[1]
    CP = ((C + 1 + 15) // 16) * 16
    NT = N
    BLK = 2048
    CHUNK = 256

    gate_i32 = gate_mask.astype(jnp.int32)
    gate3 = gate_i32[:NT].reshape(NT // BLK, 1, BLK)

    zsum = jnp.zeros((CP, D), jnp.float32)
    zcnt = jnp.zeros((CP, D), jnp.float32)
    ones256 = jnp.ones((CHUNK, D), jnp.float32)

    H = N // 2
    gate3a = gate_i32[:H].reshape(H // BLK, 1, BLK)
    gate3b = gate_i32[H:].reshape(H // BLK, 1, BLK)
    la = _compute_labels(logits, gate3a, H, BLK).reshape(H)
    segA = _make_segment_sum(H, D, CP, CHUNK)
    sumsA, cntsA = segA(la, img_feats, zsum, zcnt, ones256)
    lb = _compute_labels(logits, gate3b, H, BLK, row_off=H).reshape(H)
    segB = _make_segment_sum(H, D, CP, CHUNK, row_off=H)
    sumsB, cntsB = segB(lb, img_feats, zsum, zcnt, ones256)

    text_pad = jnp.pad(text_norm_feats, ((0, CP - C), (0, 0)))
    loss = _finalize(sumsA, sumsB, cntsA, cntsB, text_pad, C)
    return loss[0, 0]
